# baseline (speedup 1.0000x reference)
import jax
import jax.numpy as jnp
from jax import lax
from jax.experimental import pallas as pl
from jax.experimental.pallas import tpu as pltpu
from jax.experimental.pallas import tpu_sc as plsc

_B = 16384
_D = 32
_NC = 2
_NS = 16
_NW = _NC * _NS
_BPW = _B // _NW
_CH = 128
_NCH = _BPW // _CH
_L = 16


def _body(z_hbm, rows_hbm, cols_hbm, out_hbm, ridx, cidx, zr, zc, outv, sem):
    wid = lax.axis_index("s") * _NC + lax.axis_index("c")
    base = wid * _BPW

    for j in range(_NCH):
        pltpu.sync_copy(rows_hbm.at[pl.ds(base + j * _CH, _CH)], ridx.at[j])
        pltpu.sync_copy(cols_hbm.at[pl.ds(base + j * _CH, _CH)], cidx.at[j])

    copies = []
    for j in range(_NCH):
        copies.append(
            pltpu.async_copy(z_hbm.at[ridx.at[j]],
                             zr.at[pl.ds(j * _CH, _CH)], sem))
        copies.append(
            pltpu.async_copy(z_hbm.at[cidx.at[j]],
                             zc.at[pl.ds(j * _CH, _CH)], sem))
    for cp in copies:
        cp.wait()

    def group(g, carry):
        rowi = lax.iota(jnp.int32, _L) + g * _L
        acc = jnp.zeros((_L,), jnp.float32)
        for d in range(_D):
            colx = jnp.full((_L,), d, jnp.int32)
            r = plsc.load_gather(zr, [rowi, colx])
            c = plsc.load_gather(zc, [rowi, colx])
            acc = acc + r * c
        outv[pl.ds(g * _L, _L)] = acc
        return carry

    lax.fori_loop(0, _BPW // _L, group, 0)
    pltpu.sync_copy(outv, out_hbm.at[pl.ds(base, _BPW)])


def kernel(z, drp_rows, drp_cols):
    mesh = plsc.VectorSubcoreMesh(core_axis_name="c", subcore_axis_name="s")
    f = pl.kernel(
        _body,
        out_type=jax.ShapeDtypeStruct((_B,), jnp.float32),
        mesh=mesh,
        compiler_params=pltpu.CompilerParams(
            needs_layout_passes=False, use_tc_tiling_on_sc=False),
        scratch_types=[
            pltpu.VMEM((_NCH, _CH), jnp.int32),
            pltpu.VMEM((_NCH, _CH), jnp.int32),
            pltpu.VMEM((_BPW, _D), jnp.float32),
            pltpu.VMEM((_BPW, _D), jnp.float32),
            pltpu.VMEM((_BPW,), jnp.float32),
            pltpu.SemaphoreType.DMA,
        ],
    )
    return f(z, drp_rows.astype(jnp.int32), drp_cols.astype(jnp.int32))
